# baseline (speedup 1.0000x reference)
import functools

import jax
import jax.numpy as jnp
from jax import lax
from jax.experimental import pallas as pl
from jax.experimental.pallas import tpu as pltpu
from jax.experimental.pallas import tpu_sc as plsc

_VOCAB = 100000
_HIDDEN = 64
_LANG_PAD = 128
_BATCH = 4096
_SEQ = 200
_HALF = 104
_OFF2 = _SEQ - _HALF
_SEQP = 2 * _HALF

_NC = 2
_NS = 16
_NW = _NC * _NS
_RPW = _BATCH // _NW

_VBLK = 10000


def _proj_body(emb_ref, tw_ref, wt_ref, b_ref, out_ref):
    weighted = emb_ref[...] * tw_ref[...]
    out_ref[...] = (
        jnp.dot(weighted, wt_ref[...], preferred_element_type=jnp.float32)
        + b_ref[...]
    ).astype(jnp.bfloat16)


def _project_table(embeddings, token_weights, W, b):
    wt = jnp.zeros((_HIDDEN, _LANG_PAD), jnp.float32).at[:, : W.shape[0]].set(W.T)
    b2 = jnp.zeros((1, _LANG_PAD), jnp.float32).at[0, : b.shape[0]].set(b)
    return pl.pallas_call(
        _proj_body,
        grid=(_VOCAB // _VBLK,),
        in_specs=[
            pl.BlockSpec((_VBLK, _HIDDEN), lambda i: (i, 0)),
            pl.BlockSpec((_VBLK, 1), lambda i: (i, 0)),
            pl.BlockSpec((_HIDDEN, _LANG_PAD), lambda i: (0, 0)),
            pl.BlockSpec((1, _LANG_PAD), lambda i: (0, 0)),
        ],
        out_specs=pl.BlockSpec((_VBLK, _LANG_PAD), lambda i: (i, 0)),
        out_shape=jax.ShapeDtypeStruct((_VOCAB, _LANG_PAD), jnp.bfloat16),
    )(embeddings, token_weights, wt, b2)


@functools.partial(
    pl.kernel,
    mesh=plsc.VectorSubcoreMesh(core_axis_name="c", subcore_axis_name="s"),
    out_type=jax.ShapeDtypeStruct((_BATCH, _LANG_PAD), jnp.bfloat16),
    scratch_types=[
        pltpu.VMEM((_RPW, _SEQ), jnp.int32),
        pltpu.VMEM((_SEQP, _LANG_PAD), jnp.bfloat16),
        pltpu.VMEM((_SEQP, _LANG_PAD), jnp.bfloat16),
        pltpu.VMEM((_RPW, _LANG_PAD), jnp.bfloat16),
        pltpu.SemaphoreType.DMA,
        pltpu.SemaphoreType.DMA,
    ],
    compiler_params=pltpu.CompilerParams(use_tc_tiling_on_sc=False),
)
def _gather_max(ids_hbm, p_hbm, out_hbm, idx_v, buf_a, buf_b, out_v, sem_a, sem_b):
    wid = lax.axis_index("s") * _NC + lax.axis_index("c")
    base = wid * _RPW
    pltpu.sync_copy(ids_hbm.at[pl.ds(base, _RPW), :], idx_v)

    def start(r, buf, sem):
        pltpu.async_copy(
            p_hbm.at[idx_v.at[r, pl.ds(0, _HALF)]],
            buf.at[pl.ds(0, _HALF), :],
            sem,
        )
        pltpu.async_copy(
            p_hbm.at[idx_v.at[r, pl.ds(_OFF2, _HALF)]],
            buf.at[pl.ds(_HALF, _HALF), :],
            sem,
        )

    def drain(buf, sem):
        pltpu.make_async_copy(p_hbm.at[pl.ds(0, _SEQP), :], buf, sem).wait()

    def reduce_row(r, buf):
        acc = tuple(buf[0, pl.ds(j * 32, 32)] for j in range(_LANG_PAD // 32))

        def sbody(s, a):
            return tuple(
                jnp.maximum(a[j], buf[s, pl.ds(j * 32, 32)])
                for j in range(_LANG_PAD // 32)
            )

        acc = lax.fori_loop(1, _SEQP, sbody, acc)
        for j in range(_LANG_PAD // 32):
            out_v[r, pl.ds(j * 32, 32)] = acc[j]

    start(0, buf_a, sem_a)

    def pair(i, carry):
        r = 2 * i
        start(r + 1, buf_b, sem_b)
        drain(buf_a, sem_a)
        reduce_row(r, buf_a)

        @pl.when(i + 1 < _RPW // 2)
        def _():
            start(r + 2, buf_a, sem_a)

        drain(buf_b, sem_b)
        reduce_row(r + 1, buf_b)
        return carry

    lax.fori_loop(0, _RPW // 2, pair, 0)
    pltpu.sync_copy(out_v, out_hbm.at[pl.ds(base, _RPW), :])


def kernel(token_ids, embeddings, token_weights, W, b):
    p = _project_table(embeddings, token_weights, W, b)
    out = _gather_max(token_ids, p)
    return out[:, : W.shape[0]].astype(jnp.float32)


# speedup vs baseline: 1.1241x; 1.1241x over previous
import functools

import jax
import jax.numpy as jnp
from jax import lax
from jax.experimental import pallas as pl
from jax.experimental.pallas import tpu as pltpu
from jax.experimental.pallas import tpu_sc as plsc

_VOCAB = 100000
_HIDDEN = 64
_LANG_PAD = 128
_BATCH = 4096
_SEQ = 200
_HALF = 104
_OFF2 = _SEQ - _HALF
_SEQP = 2 * _HALF

_NC = 2
_NS = 16
_NW = _NC * _NS
_RPW = _BATCH // _NW

_VBLK = 10000


def _proj_body(emb_ref, tw_ref, wt_ref, b_ref, out_ref):
    weighted = emb_ref[...] * tw_ref[...]
    proj = (
        jnp.dot(weighted, wt_ref[...], preferred_element_type=jnp.float32)
        + b_ref[...]
    )
    lo = lax.bitcast_convert_type(
        proj[:, : _LANG_PAD // 2].astype(jnp.bfloat16), jnp.uint16
    ).astype(jnp.uint32)
    hi = lax.bitcast_convert_type(
        proj[:, _LANG_PAD // 2 :].astype(jnp.bfloat16), jnp.uint16
    ).astype(jnp.uint32)
    out_ref[...] = lax.bitcast_convert_type(lo | (hi << 16), jnp.int32)


def _project_table(embeddings, token_weights, W, b):
    wt = jnp.zeros((_HIDDEN, _LANG_PAD), jnp.float32).at[:, : W.shape[0]].set(W.T)
    b2 = jnp.zeros((1, _LANG_PAD), jnp.float32).at[0, : b.shape[0]].set(b)
    return pl.pallas_call(
        _proj_body,
        grid=(_VOCAB // _VBLK,),
        in_specs=[
            pl.BlockSpec((_VBLK, _HIDDEN), lambda i: (i, 0)),
            pl.BlockSpec((_VBLK, 1), lambda i: (i, 0)),
            pl.BlockSpec((_HIDDEN, _LANG_PAD), lambda i: (0, 0)),
            pl.BlockSpec((1, _LANG_PAD), lambda i: (0, 0)),
        ],
        out_specs=pl.BlockSpec((_VBLK, _LANG_PAD // 2), lambda i: (i, 0)),
        out_shape=jax.ShapeDtypeStruct((_VOCAB, _LANG_PAD // 2), jnp.int32),
    )(embeddings, token_weights, wt, b2)


@functools.partial(
    pl.kernel,
    mesh=plsc.VectorSubcoreMesh(core_axis_name="c", subcore_axis_name="s"),
    out_type=jax.ShapeDtypeStruct((_BATCH, _LANG_PAD // 2), jnp.int32),
    scratch_types=[
        pltpu.VMEM((_RPW, _SEQ), jnp.int32),
        pltpu.VMEM((_SEQP, _LANG_PAD // 2), jnp.int32),
        pltpu.VMEM((_SEQP, _LANG_PAD // 2), jnp.int32),
        pltpu.VMEM((_RPW, _LANG_PAD // 2), jnp.int32),
        pltpu.SemaphoreType.DMA,
        pltpu.SemaphoreType.DMA,
    ],
    compiler_params=pltpu.CompilerParams(
        use_tc_tiling_on_sc=False, needs_layout_passes=False
    ),
)
def _gather_max(ids_hbm, p_hbm, out_hbm, idx_v, buf_a, buf_b, out_v, sem_a, sem_b):
    wid = lax.axis_index("s") * _NC + lax.axis_index("c")
    base = wid * _RPW
    pltpu.sync_copy(ids_hbm.at[pl.ds(base, _RPW), :], idx_v)

    def start(r, buf, sem):
        pltpu.async_copy(
            p_hbm.at[idx_v.at[r, pl.ds(0, _HALF)]],
            buf.at[pl.ds(0, _HALF), :],
            sem,
        )
        pltpu.async_copy(
            p_hbm.at[idx_v.at[r, pl.ds(_OFF2, _HALF)]],
            buf.at[pl.ds(_HALF, _HALF), :],
            sem,
        )

    def drain(buf, sem):
        pltpu.make_async_copy(p_hbm.at[pl.ds(0, _SEQP), :], buf, sem).wait()

    def reduce_row(r, buf):
        nj = _LANG_PAD // 32

        def chunk(s, j):
            return plsc.bitcast(buf[s, pl.ds(j * 16, 16)], jnp.bfloat16)

        acc = tuple(chunk(0, j) for j in range(nj))

        def sbody(s, a):
            return tuple(jnp.maximum(a[j], chunk(s, j)) for j in range(nj))

        acc = lax.fori_loop(1, _SEQP, sbody, acc)
        for j in range(nj):
            out_v[r, pl.ds(j * 16, 16)] = plsc.bitcast(acc[j], jnp.int32)

    start(0, buf_a, sem_a)

    def pair(i, carry):
        r = 2 * i
        start(r + 1, buf_b, sem_b)
        drain(buf_a, sem_a)
        reduce_row(r, buf_a)

        @pl.when(i + 1 < _RPW // 2)
        def _():
            start(r + 2, buf_a, sem_a)

        drain(buf_b, sem_b)
        reduce_row(r + 1, buf_b)
        return carry

    lax.fori_loop(0, _RPW // 2, pair, 0)
    pltpu.sync_copy(out_v, out_hbm.at[pl.ds(base, _RPW), :])


def kernel(token_ids, embeddings, token_weights, W, b):
    p = _project_table(embeddings, token_weights, W, b)
    out32 = _gather_max(token_ids, p)
    pairs = lax.bitcast_convert_type(out32, jnp.uint16)
    halves = lax.bitcast_convert_type(pairs, jnp.bfloat16)
    out = jnp.concatenate([halves[..., 0], halves[..., 1]], axis=-1)
    return out[:, : W.shape[0]].astype(jnp.float32)
